# baseline (speedup 1.0000x reference)
import functools

import jax
import jax.numpy as jnp
from jax import lax
from jax.experimental import pallas as pl
from jax.experimental.pallas import tpu as pltpu

_LMBD0 = 0.001
_LMBD1 = 1.0
_LMBD2 = 0.001
_ALPHA = 1.0


def _cdiv(a, b):
    return -(-a // b)


def _contrib(lbl, u, num_classes):
    t = jnp.tanh(u)
    tn, nbit = t.shape
    tb = t.astype(jnp.bfloat16)
    rhs = jnp.concatenate([tb, tb * tb, jnp.abs(tb)], axis=1)

    iota_sub = lax.broadcasted_iota(jnp.int32, (num_classes, 128), 0)
    pieces = [
        (iota_sub == jnp.broadcast_to(lbl[s:s + 1, :], (num_classes, 128)))
        for s in range(tn // 128)
    ]
    lhs2 = jnp.concatenate(
        [p.astype(jnp.bfloat16) for p in pieces], axis=1)

    dot = lax.dot_general(
        lhs2, rhs, (((1,), (0,)), ((), ())),
        preferred_element_type=jnp.float32)

    pcount = pieces[0].astype(jnp.float32)
    for p in pieces[1:]:
        pcount += p.astype(jnp.float32)
    return dot, pcount


def _finalize(agg, wf, n, nbit):
    m = agg[:, :nbit]
    t_c = jnp.sum(agg[:, nbit:2 * nbit], axis=1, keepdims=True)
    sum_abs = jnp.sum(agg[:, 2 * nbit:3 * nbit])
    counts = jnp.sum(agg[:, 3 * nbit:], axis=1, keepdims=True)
    ucol = jnp.sum(m, axis=0, keepdims=True)
    sum_uu = jnp.sum(t_c)

    n_f = float(n)
    j1 = (sum_uu - 2.0 * sum_abs + n_f * float(nbit)) / n_f
    posn = jnp.sum(counts * counts)
    negn = n_f * n_f - posn
    s_pos = 2.0 * jnp.sum(counts * t_c) - 2.0 * jnp.sum(m * m)
    s_all = 2.0 * n_f * sum_uu - 2.0 * jnp.sum(ucol * ucol)
    s_neg = s_all - s_pos
    j2_2 = s_neg / (negn + 1e-7) - s_pos / (posn + 1e-7)
    j2 = (sum_uu + _ALPHA * j2_2) / 2.0

    ortho = lax.dot_general(wf, wf, (((1,), (1,)), ((), ())),
                            preferred_element_type=jnp.float32)
    j3 = jnp.sum((ortho - 1.0) ** 2) / 2.0

    return _LMBD0 * j1 - _LMBD1 * j2 + _LMBD2 * j3


def _sdh_kernel(*refs, num_classes, nstream, n, nbit):
    lbl_refs = refs[:nstream]
    u_refs = refs[nstream:2 * nstream]
    w_ref = refs[2 * nstream]
    out_ref = refs[2 * nstream + 1]
    acc_ref = refs[2 * nstream + 2]
    r = pl.program_id(0)

    @pl.when(r == 0)
    def _():
        acc_ref[...] = jnp.zeros_like(acc_ref)

    dot, pcount = _contrib(lbl_refs[0][...], u_refs[0][...], num_classes)
    for k in range(1, nstream):
        d, p = _contrib(lbl_refs[k][...], u_refs[k][...], num_classes)
        dot += d
        pcount += p
    acc_ref[:, :3 * nbit] += dot
    acc_ref[:, 3 * nbit:] += pcount

    @pl.when(r == pl.num_programs(0) - 1)
    def _():
        loss = _finalize(acc_ref[...], w_ref[...], n, nbit)
        out_ref[...] = jnp.reshape(loss, (1, 1))


def kernel(u, labels, w):
    n, nbit = u.shape
    c = w.shape[1]
    tn = 2048
    nstream = 16
    rows_per_step = tn * nstream
    r_blocks = _cdiv(n, rows_per_step)
    padded = r_blocks * rows_per_step

    lbl_flat = labels.reshape(n).astype(jnp.int32)
    u_in = u
    if padded != n:
        u_in = jnp.zeros((padded, nbit), u.dtype).at[:n].set(u)
        lbl_flat = jnp.full((padded,), c, jnp.int32).at[:n].set(lbl_flat)
    lbl2 = lbl_flat.reshape(padded // 128, 128)

    width = 3 * nbit + 128
    lbl_specs = [
        pl.BlockSpec((tn // 128, 128), lambda r, kk=k: (r * nstream + kk, 0))
        for k in range(nstream)
    ]
    u_specs = [
        pl.BlockSpec((tn, nbit), lambda r, kk=k: (r * nstream + kk, 0))
        for k in range(nstream)
    ]
    out = pl.pallas_call(
        functools.partial(_sdh_kernel, num_classes=c, nstream=nstream,
                          n=n, nbit=nbit),
        out_shape=jax.ShapeDtypeStruct((1, 1), jnp.float32),
        grid_spec=pltpu.PrefetchScalarGridSpec(
            num_scalar_prefetch=0,
            grid=(r_blocks,),
            in_specs=lbl_specs + u_specs
            + [pl.BlockSpec((nbit, c), lambda r: (0, 0))],
            out_specs=pl.BlockSpec((1, 1), lambda r: (0, 0)),
            scratch_shapes=[pltpu.VMEM((c, width), jnp.float32)],
        ),
        compiler_params=pltpu.CompilerParams(
            dimension_semantics=("arbitrary",),
            vmem_limit_bytes=64 * 1024 * 1024,
        ),
    )(*([lbl2] * nstream + [u_in] * nstream + [w.astype(jnp.float32)]))
    return out[0, 0]


# speedup vs baseline: 1.0746x; 1.0746x over previous
import functools

import jax
import jax.numpy as jnp
from jax import lax
from jax.experimental import pallas as pl
from jax.experimental.pallas import tpu as pltpu

_LMBD0 = 0.001
_LMBD1 = 1.0
_LMBD2 = 0.001
_ALPHA = 1.0


def _cdiv(a, b):
    return -(-a // b)


def _contrib(lbl, u, num_classes):
    t = jnp.tanh(u)
    tn, nbit = t.shape
    tb = t.astype(jnp.bfloat16)
    rhs = jnp.concatenate([tb, tb * tb, jnp.abs(tb)], axis=1)

    iota_sub = lax.broadcasted_iota(jnp.int32, (num_classes, 128), 0)
    pieces = [
        (iota_sub == jnp.broadcast_to(lbl[s:s + 1, :], (num_classes, 128)))
        for s in range(tn // 128)
    ]
    pieces_bf = [p.astype(jnp.bfloat16) for p in pieces]
    lhs2 = jnp.concatenate(pieces_bf, axis=1)

    dot = lax.dot_general(
        lhs2, rhs, (((1,), (0,)), ((), ())),
        preferred_element_type=jnp.float32)

    pcount = pieces_bf[0]
    for p in pieces_bf[1:]:
        pcount = pcount + p
    return dot, pcount.astype(jnp.float32)


def _finalize(agg, wf, n, nbit):
    m = agg[:, :nbit]
    t_c = jnp.sum(agg[:, nbit:2 * nbit], axis=1, keepdims=True)
    sum_abs = jnp.sum(agg[:, 2 * nbit:3 * nbit])
    counts = jnp.sum(agg[:, 3 * nbit:], axis=1, keepdims=True)
    ucol = jnp.sum(m, axis=0, keepdims=True)
    sum_uu = jnp.sum(t_c)

    n_f = float(n)
    j1 = (sum_uu - 2.0 * sum_abs + n_f * float(nbit)) / n_f
    posn = jnp.sum(counts * counts)
    negn = n_f * n_f - posn
    s_pos = 2.0 * jnp.sum(counts * t_c) - 2.0 * jnp.sum(m * m)
    s_all = 2.0 * n_f * sum_uu - 2.0 * jnp.sum(ucol * ucol)
    s_neg = s_all - s_pos
    j2_2 = (s_neg / jnp.maximum(negn + 1e-7, 1e-7)
            - s_pos / jnp.maximum(posn + 1e-7, 1e-7))
    j2 = (sum_uu + _ALPHA * j2_2) / 2.0

    ortho = lax.dot_general(wf, wf, (((1,), (1,)), ((), ())),
                            preferred_element_type=jnp.float32)
    j3 = jnp.sum((ortho - 1.0) ** 2) / 2.0

    return _LMBD0 * j1 - _LMBD1 * j2 + _LMBD2 * j3


def _sdh_kernel(*refs, num_classes, nstream, n, nbit):
    lbl_refs = refs[:nstream]
    u_refs = refs[nstream:2 * nstream]
    w_ref = refs[2 * nstream]
    out_ref = refs[2 * nstream + 1]
    acc_ref = refs[2 * nstream + 2]
    r = pl.program_id(0)

    @pl.when(r == 0)
    def _():
        acc_ref[...] = jnp.zeros_like(acc_ref)

    dot, pcount = _contrib(lbl_refs[0][...], u_refs[0][...], num_classes)
    for k in range(1, nstream):
        d, p = _contrib(lbl_refs[k][...], u_refs[k][...], num_classes)
        dot += d
        pcount += p
    acc_ref[:, :3 * nbit] += dot
    acc_ref[:, 3 * nbit:] += pcount

    @pl.when(r == pl.num_programs(0) - 1)
    def _():
        loss = _finalize(acc_ref[...], w_ref[...], n, nbit)
        out_ref[...] = jnp.reshape(loss, (1, 1))


def kernel(u, labels, w):
    n, nbit = u.shape
    c = w.shape[1]
    tn = 2048
    nstream = 8
    rows_per_step = tn * nstream
    r_blocks = _cdiv(n, rows_per_step)
    padded = r_blocks * rows_per_step

    lbl_flat = labels.reshape(n).astype(jnp.int32)
    u_in = u
    if padded != n:
        u_in = jnp.zeros((padded, nbit), u.dtype).at[:n].set(u)
        lbl_flat = jnp.full((padded,), c, jnp.int32).at[:n].set(lbl_flat)
    lbl2 = lbl_flat.reshape(padded // 128, 128)

    width = 3 * nbit + 128
    lbl_specs = [
        pl.BlockSpec((tn // 128, 128), lambda r, kk=k: (r * nstream + kk, 0))
        for k in range(nstream)
    ]
    u_specs = [
        pl.BlockSpec((tn, nbit), lambda r, kk=k: (r * nstream + kk, 0))
        for k in range(nstream)
    ]
    out = pl.pallas_call(
        functools.partial(_sdh_kernel, num_classes=c, nstream=nstream,
                          n=n, nbit=nbit),
        out_shape=jax.ShapeDtypeStruct((1, 1), jnp.float32),
        grid_spec=pltpu.PrefetchScalarGridSpec(
            num_scalar_prefetch=0,
            grid=(r_blocks,),
            in_specs=lbl_specs + u_specs
            + [pl.BlockSpec((nbit, c), lambda r: (0, 0))],
            out_specs=pl.BlockSpec((1, 1), lambda r: (0, 0)),
            scratch_shapes=[pltpu.VMEM((c, width), jnp.float32)],
        ),
        compiler_params=pltpu.CompilerParams(
            dimension_semantics=("arbitrary",),
            vmem_limit_bytes=64 * 1024 * 1024,
        ),
    )(*([lbl2] * nstream + [u_in] * nstream + [w.astype(jnp.float32)]))
    return out[0, 0]


# speedup vs baseline: 1.0823x; 1.0071x over previous
import functools

import jax
import jax.numpy as jnp
from jax import lax
from jax.experimental import pallas as pl
from jax.experimental.pallas import tpu as pltpu

_LMBD0 = 0.001
_LMBD1 = 1.0
_LMBD2 = 0.001
_ALPHA = 1.0


def _cdiv(a, b):
    return -(-a // b)


def _contrib(lbl, u, num_classes):
    t = jnp.tanh(u)
    tn, nbit = t.shape
    tb = t.astype(jnp.bfloat16)
    ones = jnp.ones((tn, 128), jnp.bfloat16)
    rhs = jnp.concatenate([tb, tb * tb, jnp.abs(tb), ones], axis=1)

    iota_sub = lax.broadcasted_iota(jnp.int32, (num_classes, 128), 0)
    pieces = [
        (iota_sub == jnp.broadcast_to(lbl[s:s + 1, :], (num_classes, 128)))
        for s in range(tn // 128)
    ]
    pieces_bf = [p.astype(jnp.bfloat16) for p in pieces]
    lhs2 = jnp.concatenate(pieces_bf, axis=1)

    return lax.dot_general(
        lhs2, rhs, (((1,), (0,)), ((), ())),
        preferred_element_type=jnp.float32)


def _finalize(agg, wf, n, nbit):
    m = agg[:, :nbit]
    t_c = jnp.sum(agg[:, nbit:2 * nbit], axis=1, keepdims=True)
    sum_abs = jnp.sum(agg[:, 2 * nbit:3 * nbit])
    counts = agg[:, 3 * nbit:3 * nbit + 1]
    ucol = jnp.sum(m, axis=0, keepdims=True)
    sum_uu = jnp.sum(t_c)

    n_f = float(n)
    j1 = (sum_uu - 2.0 * sum_abs + n_f * float(nbit)) / n_f
    posn = jnp.sum(counts * counts)
    negn = n_f * n_f - posn
    s_pos = 2.0 * jnp.sum(counts * t_c) - 2.0 * jnp.sum(m * m)
    s_all = 2.0 * n_f * sum_uu - 2.0 * jnp.sum(ucol * ucol)
    s_neg = s_all - s_pos
    j2_2 = (s_neg / jnp.maximum(negn + 1e-7, 1e-7)
            - s_pos / jnp.maximum(posn + 1e-7, 1e-7))
    j2 = (sum_uu + _ALPHA * j2_2) / 2.0

    ortho = lax.dot_general(wf, wf, (((1,), (1,)), ((), ())),
                            preferred_element_type=jnp.float32)
    j3 = jnp.sum((ortho - 1.0) ** 2) / 2.0

    return _LMBD0 * j1 - _LMBD1 * j2 + _LMBD2 * j3


def _sdh_kernel(*refs, num_classes, nstream, n, nbit):
    lbl_refs = refs[:nstream]
    u_refs = refs[nstream:2 * nstream]
    w_ref = refs[2 * nstream]
    out_ref = refs[2 * nstream + 1]
    acc_ref = refs[2 * nstream + 2]
    r = pl.program_id(0)

    @pl.when(r == 0)
    def _():
        acc_ref[...] = jnp.zeros_like(acc_ref)

    dot = _contrib(lbl_refs[0][...], u_refs[0][...], num_classes)
    for k in range(1, nstream):
        dot += _contrib(lbl_refs[k][...], u_refs[k][...], num_classes)
    acc_ref[...] += dot

    @pl.when(r == pl.num_programs(0) - 1)
    def _():
        loss = _finalize(acc_ref[...], w_ref[...], n, nbit)
        out_ref[...] = jnp.reshape(loss, (1, 1))


def kernel(u, labels, w):
    n, nbit = u.shape
    c = w.shape[1]
    tn = 2048
    nstream = 8
    rows_per_step = tn * nstream
    r_blocks = _cdiv(n, rows_per_step)
    padded = r_blocks * rows_per_step

    lbl_flat = labels.reshape(n).astype(jnp.int32)
    u_in = u
    if padded != n:
        u_in = jnp.zeros((padded, nbit), u.dtype).at[:n].set(u)
        lbl_flat = jnp.full((padded,), c, jnp.int32).at[:n].set(lbl_flat)
    lbl2 = lbl_flat.reshape(padded // 128, 128)

    width = 3 * nbit + 128
    lbl_specs = [
        pl.BlockSpec((tn // 128, 128), lambda r, kk=k: (r * nstream + kk, 0))
        for k in range(nstream)
    ]
    u_specs = [
        pl.BlockSpec((tn, nbit), lambda r, kk=k: (r * nstream + kk, 0))
        for k in range(nstream)
    ]
    out = pl.pallas_call(
        functools.partial(_sdh_kernel, num_classes=c, nstream=nstream,
                          n=n, nbit=nbit),
        out_shape=jax.ShapeDtypeStruct((1, 1), jnp.float32),
        grid_spec=pltpu.PrefetchScalarGridSpec(
            num_scalar_prefetch=0,
            grid=(r_blocks,),
            in_specs=lbl_specs + u_specs
            + [pl.BlockSpec((nbit, c), lambda r: (0, 0))],
            out_specs=pl.BlockSpec((1, 1), lambda r: (0, 0)),
            scratch_shapes=[pltpu.VMEM((c, width), jnp.float32)],
        ),
        compiler_params=pltpu.CompilerParams(
            dimension_semantics=("arbitrary",),
            vmem_limit_bytes=64 * 1024 * 1024,
        ),
    )(*([lbl2] * nstream + [u_in] * nstream + [w.astype(jnp.float32)]))
    return out[0, 0]
